# baseline (speedup 1.0000x reference)
import functools

import jax
import jax.numpy as jnp
from jax import lax
from jax.experimental import pallas as pl
from jax.experimental.pallas import tpu as pltpu
from jax.experimental.pallas import tpu_sc as plsc

_ALPHA = 1e-05
_C = 44
_H = 224
_W = 224
_B = 8
_R = 224
_LN2 = 0.6931471805599453

_NW = 32
_WPB = _NW // _B
_HCHUNK = _H // _WPB


def _tc_body(f_ref, m_ref, rows_ref):
    del m_ref
    for rc in range(_R // 8):
        r0 = rc * 8
        accs = [None, None, None, None]
        for c in range(_C):
            g = jnp.tanh(f_ref[0, c, r0 : r0 + 8, :])
            i = c % 4
            accs[i] = g if accs[i] is None else accs[i] + g
        s = (accs[0] + accs[1]) + (accs[2] + accs[3])
        q = 0.5 / jnp.maximum((s + _C) * 0.5, 1e-12)
        eaccs = [None, None, None, None]
        for c in range(_C):
            fn = jnp.tanh(f_ref[0, c, r0 : r0 + 8, :]) * q + q
            e = fn * jnp.log2(fn + 1e-4)
            i = c % 4
            eaccs[i] = e if eaccs[i] is None else eaccs[i] + e
        rows_ref[0, r0 : r0 + 8, :] = (eaccs[0] + eaccs[1]) + (eaccs[2] + eaccs[3])


def _sc_reduce_make():
    mesh = plsc.VectorSubcoreMesh(core_axis_name="c", subcore_axis_name="s")

    @functools.partial(
        pl.kernel,
        mesh=mesh,
        out_type=jax.ShapeDtypeStruct((2, _NW, 16), jnp.float32),
        scratch_types=[
            pltpu.VMEM((_HCHUNK, _W), jnp.float32),
            pltpu.VMEM((_HCHUNK, _W), jnp.int32),
            pltpu.VMEM((2, 16), jnp.float32),
        ],
    )
    def k(re_hbm, m_hbm, out_hbm, re_v, m_v, acc_v):
        wid = lax.axis_index("s") * 2 + lax.axis_index("c")
        b = wid // _WPB
        h0 = (wid % _WPB) * _HCHUNK
        pltpu.sync_copy(re_hbm.at[b, pl.ds(h0, _HCHUNK), :], re_v)
        pltpu.sync_copy(m_hbm.at[b, pl.ds(h0, _HCHUNK), :], m_v)

        def row_body(r, carry):
            ae, ac = carry
            for j in range(_W // 16):
                re = re_v[r, pl.ds(j * 16, 16)]
                mm = m_v[r, pl.ds(j * 16, 16)]
                sel = mm == 1
                ae = ae + jnp.where(sel, re, 0.0)
                ac = ac + jnp.where(sel, 1.0, 0.0)
            return ae, ac

        z = jnp.zeros((16,), jnp.float32)
        ae, ac = lax.fori_loop(0, _HCHUNK, row_body, (z, z))
        acc_v[0, :] = ae
        acc_v[1, :] = ac
        pltpu.sync_copy(acc_v.at[0], out_hbm.at[0, wid])
        pltpu.sync_copy(acc_v.at[1], out_hbm.at[1, wid])

    return k


@jax.jit
def kernel(feature, mask):
    rows = pl.pallas_call(
        _tc_body,
        grid=(_B, 1),
        in_specs=[
            pl.BlockSpec((1, _C, _R, _W), lambda b, t: (b, 0, t, 0)),
            pl.BlockSpec((1, _R, _W), lambda b, t: (b, t, 0)),
        ],
        out_specs=pl.BlockSpec((1, _H, _W), lambda b, t: (b, t, 0)),
        out_shape=jax.ShapeDtypeStruct((_B, _H, _W), jnp.float32),
    )(feature, mask)

    parts = _sc_reduce_make()(rows, mask)

    ent_sum = -jnp.sum(parts[0]) * _LN2
    cnt = jnp.sum(parts[1])
    loss = _ALPHA * ent_sum / (_C * jnp.maximum(cnt, 1.0))
    return jnp.where(cnt == 0.0, jnp.float32(0.0), loss.astype(jnp.float32))


# speedup vs baseline: 1.5774x; 1.5774x over previous
import jax
import jax.numpy as jnp
from jax.experimental import pallas as pl

_ALPHA = 1e-05
_C = 44
_H = 224
_W = 224
_B = 8
_R = 224
_LN2 = 0.6931471805599453


def _body(f_ref, m_ref, out_ref):
    b = pl.program_id(0)
    t = pl.program_id(1)

    pe = jnp.zeros((8, _W), jnp.float32)
    pc = jnp.zeros((8, _W), jnp.float32)
    for rc in range(_R // 8):
        r0 = rc * 8
        accs = [None, None, None, None]
        gs = []
        for c in range(_C):
            g = jnp.tanh(f_ref[0, c, r0 : r0 + 8, :])
            gs.append(g)
            i = c % 4
            accs[i] = g if accs[i] is None else accs[i] + g
        s = (accs[0] + accs[1]) + (accs[2] + accs[3])
        q = 0.5 / jnp.maximum((s + _C) * 0.5, 1e-12)
        eaccs = [None, None, None, None]
        for c in range(_C):
            fn = gs[c] * q + q
            e = fn * jnp.log2(fn + 1e-4)
            i = c % 4
            eaccs[i] = e if eaccs[i] is None else eaccs[i] + e
        row = (eaccs[0] + eaccs[1]) + (eaccs[2] + eaccs[3])
        msel = m_ref[0, r0 : r0 + 8, :] == 1
        pe = pe + jnp.where(msel, row, 0.0)
        pc = pc + msel.astype(jnp.float32)

    part_ent = jnp.sum(pe)
    part_cnt = jnp.sum(pc)

    lane = jax.lax.broadcasted_iota(jnp.int32, (1, 128), 1)
    v = jnp.where(lane == 0, part_ent, 0.0) + jnp.where(lane == 1, part_cnt, 0.0)

    @pl.when(jnp.logical_and(b == 0, t == 0))
    def _init():
        out_ref[...] = jnp.zeros_like(out_ref)

    out_ref[...] += v


@jax.jit
def kernel(feature, mask):
    grid = (_B, _H // _R)
    out = pl.pallas_call(
        _body,
        grid=grid,
        in_specs=[
            pl.BlockSpec((1, _C, _R, _W), lambda b, t: (b, 0, t, 0)),
            pl.BlockSpec((1, _R, _W), lambda b, t: (b, t, 0)),
        ],
        out_specs=pl.BlockSpec((1, 128), lambda b, t: (0, 0)),
        out_shape=jax.ShapeDtypeStruct((1, 128), jnp.float32),
    )(feature, mask)

    ent_sum = -out[0, 0] * _LN2
    cnt = out[0, 1]
    loss = _ALPHA * ent_sum / (_C * jnp.maximum(cnt, 1.0))
    return jnp.where(cnt == 0.0, jnp.float32(0.0), loss.astype(jnp.float32))
